# Initial kernel scaffold; baseline (speedup 1.0000x reference)
import jax
import jax.numpy as jnp
from jax.experimental import pallas as pl


def kernel(xyz1, xyz2, flow1):
    raise NotImplementedError("write your pallas kernel here")


# speedup vs baseline: 32.5488x; 32.5488x over previous
import functools

import jax
import jax.numpy as jnp
from jax.experimental import pallas as pl

TQ = 256


def _warp_kernel(q_ref, k_ref, f_ref, o_ref, *, n1):
    q = q_ref[0]
    k = k_ref[0]
    fl = f_ref[0]

    mm = jnp.dot(q, k)
    qn = jnp.sum(q * q, axis=1, keepdims=True)
    kn = jnp.sum(k * k, axis=0, keepdims=True)
    d = -2.0 * mm
    d = d + qn
    d = d + kn

    iota = jax.lax.broadcasted_iota(jnp.int32, (d.shape[0], n1), 1)
    selmask = jnp.zeros(d.shape, dtype=jnp.bool_)
    for _ in range(3):
        m = jnp.min(d, axis=1, keepdims=True)
        first = jnp.min(jnp.where(d == m, iota, n1), axis=1, keepdims=True)
        hit = iota == first
        selmask = jnp.logical_or(selmask, hit)
        d = jnp.where(hit, jnp.inf, d)

    q0, q1, q2 = q[:, 0:1], q[:, 1:2], q[:, 2:3]
    k0, k1, k2 = k[0:1, :], k[1:2, :], k[2:3, :]
    dd = (k0 - q0) ** 2
    dd = dd + (k1 - q1) ** 2
    dd = dd + (k2 - q2) ** 2
    dist = jnp.maximum(jnp.sqrt(dd), 1e-10)
    w = jnp.where(selmask, 1.0 / dist, 0.0)
    norm = jnp.sum(w, axis=1, keepdims=True)

    f0 = jnp.sum(w * fl[0:1, :], axis=1, keepdims=True) / norm
    f1 = jnp.sum(w * fl[1:2, :], axis=1, keepdims=True) / norm
    f2 = jnp.sum(w * fl[2:3, :], axis=1, keepdims=True) / norm

    o_ref[0] = jnp.concatenate([q0 - f0, q1 - f1, q2 - f2], axis=1)


def kernel(xyz1, xyz2, flow1):
    b, c, n1 = xyz1.shape
    n2 = xyz2.shape[2]
    keys = xyz1 + flow1
    queries = jnp.transpose(xyz2, (0, 2, 1))

    out = pl.pallas_call(
        functools.partial(_warp_kernel, n1=n1),
        grid=(b, n2 // TQ),
        in_specs=[
            pl.BlockSpec((1, TQ, c), lambda i, j: (i, j, 0)),
            pl.BlockSpec((1, c, n1), lambda i, j: (i, 0, 0)),
            pl.BlockSpec((1, c, n1), lambda i, j: (i, 0, 0)),
        ],
        out_specs=pl.BlockSpec((1, TQ, c), lambda i, j: (i, j, 0)),
        out_shape=jax.ShapeDtypeStruct((b, n2, c), jnp.float32),
    )(queries, keys, flow1)

    return jnp.transpose(out, (0, 2, 1))
